# baseline (speedup 1.0000x reference)
import functools

import jax
import jax.numpy as jnp
from jax import lax
from jax.experimental import pallas as pl
from jax.experimental.pallas import tpu as pltpu
from jax.experimental.pallas import tpu_sc as plsc

B, D, T = 32, 64, 576
K = 1024
N = B * T
EPS = 1e-12
BB = 8
W = BB * T
C = D + 8
INV_ND = 1.0 / float(N * D)

NC, NS = 2, 16
NW = NC * NS
TOK_PER_W = N // NW
GCHUNK = 96


def _vq_kernel(x_ref, emb_ref, embu_ref, aux_ref, idx_ref, loss_ref, cat_ref):
    g = pl.program_id(0)

    @pl.when(g == 0)
    def _init():
        loss_ref[...] = jnp.zeros((1, 1), jnp.float32)
        cat_ref[:, :D] = embu_ref[...].astype(jnp.bfloat16)
        cat_ref[:, D:] = aux_ref[...]

    x = jnp.concatenate([x_ref[i] for i in range(BB)], axis=1)
    emb = emb_ref[...]

    norm = jnp.sqrt(jnp.sum(x * x, axis=0, keepdims=True))
    xn = x / jnp.maximum(norm, EPS)

    dist = lax.dot_general(
        emb, xn, (((1,), (0,)), ((), ())),
        preferred_element_type=jnp.float32,
    )

    maxval = jnp.max(dist, axis=0, keepdims=True)
    mask = (dist >= maxval).astype(jnp.bfloat16)

    combo = lax.dot_general(
        cat_ref[...], mask, (((0,), (0,)), ((), ())),
        preferred_element_type=jnp.float32,
    )
    idxf = combo[D] * 32.0 + combo[D + 1]
    cnt = combo[D + 2]

    idx = idxf.astype(jnp.int32)
    for i in range(BB):
        idx_ref[i, 0] = idx[i * T:(i + 1) * T]
    diff = x - combo[:D]
    loss_ref[...] += (jnp.sum(diff * diff) * INV_ND).reshape(1, 1)

    tie = jnp.max(cnt) > 1.5

    @pl.when(tie)
    def _exact():
        iota_f = lax.broadcasted_iota(jnp.int32, (K, W), 0).astype(jnp.float32)
        idxe = jnp.min(jnp.where(dist >= maxval, iota_f, float(K)), axis=0)
        idxi = idxe.astype(jnp.int32)
        for i in range(BB):
            idx_ref[i, 0] = idxi[i * T:(i + 1) * T]


def _sc_gather(table_hbm, idx_hbm, out_hbm, idx_v, rows_v, sem):
    wid = lax.axis_index("s") * NC + lax.axis_index("c")
    base = wid * TOK_PER_W
    pltpu.sync_copy(idx_hbm.at[pl.ds(base, TOK_PER_W)], idx_v)
    copies = []
    for j in range(TOK_PER_W // GCHUNK):
        copies.append(pltpu.async_copy(
            table_hbm.at[idx_v.at[pl.ds(j * GCHUNK, GCHUNK)]],
            rows_v.at[pl.ds(j * GCHUNK, GCHUNK)],
            sem,
        ))
    for c in copies:
        c.wait()
    pltpu.sync_copy(rows_v, out_hbm.at[pl.ds(base, TOK_PER_W)])


def kernel(inputs, embedding, embedding_unnormalized):
    k_iota = jnp.arange(K, dtype=jnp.int32)
    aux = jnp.stack(
        [(k_iota >> 5).astype(jnp.bfloat16),
         (k_iota & 31).astype(jnp.bfloat16),
         jnp.ones((K,), jnp.bfloat16)]
        + [jnp.zeros((K,), jnp.bfloat16)] * 5,
        axis=1,
    )

    idx3, loss_sum = pl.pallas_call(
        _vq_kernel,
        grid=(B // BB,),
        in_specs=[
            pl.BlockSpec((BB, D, T), lambda g: (g, 0, 0)),
            pl.BlockSpec((K, D), lambda g: (0, 0)),
            pl.BlockSpec((K, D), lambda g: (0, 0)),
            pl.BlockSpec((K, 8), lambda g: (0, 0)),
        ],
        out_specs=[
            pl.BlockSpec((BB, 1, T), lambda g: (g, 0, 0)),
            pl.BlockSpec((1, 1), lambda g: (0, 0)),
        ],
        out_shape=[
            jax.ShapeDtypeStruct((B, 1, T), jnp.int32),
            jax.ShapeDtypeStruct((1, 1), jnp.float32),
        ],
        scratch_shapes=[pltpu.VMEM((K, C), jnp.bfloat16)],
    )(inputs, embedding, embedding_unnormalized, aux)

    encoding_indices = idx3.reshape(N)

    mesh = plsc.VectorSubcoreMesh(core_axis_name="c", subcore_axis_name="s")
    gather = functools.partial(
        pl.kernel,
        mesh=mesh,
        out_type=jax.ShapeDtypeStruct((N, 128), jnp.float32),
        scratch_types=[
            pltpu.VMEM((TOK_PER_W,), jnp.int32),
            pltpu.VMEM((TOK_PER_W, 128), jnp.float32),
            pltpu.SemaphoreType.DMA,
        ],
    )(_sc_gather)
    table = jnp.pad(embedding, ((0, 0), (0, 128 - D)))
    rows = gather(table, encoding_indices)

    quant = rows[:, :D].reshape(B, T, D).transpose(0, 2, 1)
    loss = loss_sum.reshape(())
    return (quant, loss, loss, encoding_indices)


# speedup vs baseline: 2.1888x; 2.1888x over previous
import jax
import jax.numpy as jnp
from jax import lax
from jax.experimental import pallas as pl
from jax.experimental.pallas import tpu as pltpu

B, D, T = 32, 64, 576
K = 1024
N = B * T
EPS = 1e-12
BB = 8
W = BB * T
C = D + 8
INV_ND = 1.0 / float(N * D)


def _vq_kernel(x_ref, emb_ref, aux_ref, quant_ref, idx_ref, loss_ref, cat_ref):
    g = pl.program_id(0)

    @pl.when(g == 0)
    def _init():
        loss_ref[...] = jnp.zeros((1, 1), jnp.float32)
        cat_ref[:, :D] = emb_ref[...].astype(jnp.bfloat16)
        cat_ref[:, D:] = aux_ref[...]

    x = jnp.concatenate([x_ref[i] for i in range(BB)], axis=1)
    emb = emb_ref[...]

    nrm2 = jnp.sum(x * x, axis=0, keepdims=True)
    nrm = jnp.sqrt(nrm2)
    xn = x / jnp.maximum(nrm, EPS)

    dist = lax.dot_general(
        emb, xn, (((1,), (0,)), ((), ())),
        preferred_element_type=jnp.float32,
    )

    maxval = jnp.max(dist, axis=0, keepdims=True)
    mask = (dist >= maxval).astype(jnp.bfloat16)

    combo = lax.dot_general(
        cat_ref[...], mask, (((0,), (0,)), ((), ())),
        preferred_element_type=jnp.float32,
    )
    idxf = combo[D] * 32.0 + combo[D + 1]
    cnt = combo[D + 2]
    unorm = combo[D + 3] + combo[D + 4]

    idx = idxf.astype(jnp.int32)
    for i in range(BB):
        idx_ref[i, 0] = idx[i * T:(i + 1) * T]
        quant_ref[i] = combo[:D, i * T:(i + 1) * T]
    part = jnp.sum(nrm2[0] - 2.0 * maxval[0] * nrm[0] * unorm + unorm * unorm)
    loss_ref[...] += (part * INV_ND).reshape(1, 1)

    tie = jnp.max(cnt) > 1.5

    @pl.when(tie)
    def _exact():
        iota_f = lax.broadcasted_iota(jnp.int32, (K, W), 0).astype(jnp.float32)
        idxe = jnp.min(jnp.where(dist >= maxval, iota_f, float(K)), axis=0)
        idxi = idxe.astype(jnp.int32)
        for i in range(BB):
            idx_ref[i, 0] = idxi[i * T:(i + 1) * T]


def kernel(inputs, embedding, embedding_unnormalized):
    k_iota = jnp.arange(K, dtype=jnp.int32)
    unorm = jnp.linalg.norm(embedding_unnormalized, axis=1)
    u_hi = unorm.astype(jnp.bfloat16)
    u_lo = (unorm - u_hi.astype(jnp.float32)).astype(jnp.bfloat16)
    aux = jnp.stack(
        [(k_iota >> 5).astype(jnp.bfloat16),
         (k_iota & 31).astype(jnp.bfloat16),
         jnp.ones((K,), jnp.bfloat16),
         u_hi, u_lo]
        + [jnp.zeros((K,), jnp.bfloat16)] * 3,
        axis=1,
    )

    quant, idx3, loss_sum = pl.pallas_call(
        _vq_kernel,
        grid=(B // BB,),
        in_specs=[
            pl.BlockSpec((BB, D, T), lambda g: (g, 0, 0)),
            pl.BlockSpec((K, D), lambda g: (0, 0)),
            pl.BlockSpec((K, 8), lambda g: (0, 0)),
        ],
        out_specs=[
            pl.BlockSpec((BB, D, T), lambda g: (g, 0, 0)),
            pl.BlockSpec((BB, 1, T), lambda g: (g, 0, 0)),
            pl.BlockSpec((1, 1), lambda g: (0, 0)),
        ],
        out_shape=[
            jax.ShapeDtypeStruct((B, D, T), jnp.float32),
            jax.ShapeDtypeStruct((B, 1, T), jnp.int32),
            jax.ShapeDtypeStruct((1, 1), jnp.float32),
        ],
        scratch_shapes=[pltpu.VMEM((K, C), jnp.bfloat16)],
    )(inputs, embedding, aux)

    loss = loss_sum.reshape(())
    encoding_indices = idx3.reshape(N)
    return (quant, loss, loss, encoding_indices)
